# baseline (speedup 1.0000x reference)
import functools

import jax
import jax.numpy as jnp
from jax import lax
from jax.experimental import pallas as pl
from jax.experimental.pallas import tpu as pltpu, tpu_sc as plsc

BATCH = 16384
N_CAT = 26
N_DENSE = 13
NW = 32
ROWS_W = BATCH // NW
IDX_W = ROWS_W * N_CAT
CH = ROWS_W // 16
DEN_W = ROWS_W * N_DENSE

_mesh = plsc.VectorSubcoreMesh(core_axis_name="c", subcore_axis_name="s")


@functools.partial(
    pl.kernel,
    out_type=jax.ShapeDtypeStruct((BATCH,), jnp.float32),
    mesh=_mesh,
    compiler_params=pltpu.CompilerParams(
        use_tc_tiling_on_sc=False, needs_layout_passes=False
    ),
    scratch_types=[
        pltpu.VMEM((IDX_W,), jnp.int32),
        pltpu.VMEM((IDX_W, 1), jnp.float32),
        pltpu.VMEM((DEN_W,), jnp.float32),
        pltpu.VMEM((N_CAT + N_DENSE + 1, 16), jnp.float32),
        pltpu.VMEM((ROWS_W,), jnp.float32),
        pltpu.SemaphoreType.DMA,
    ],
)
def _criteo_sc(idx_hbm, dense_hbm, table_hbm, wb_hbm, out_hbm,
               idx_v, vals_v, dense_v, wb_v, out_v, sem):
    wid = lax.axis_index("s") * 2 + lax.axis_index("c")

    pltpu.sync_copy(wb_hbm, wb_v)
    pltpu.sync_copy(idx_hbm.at[pl.ds(wid * IDX_W, IDX_W)], idx_v)
    pltpu.sync_copy(dense_hbm.at[pl.ds(wid * DEN_W, DEN_W)], dense_v)

    pltpu.async_copy(table_hbm.at[idx_v], vals_v, sem).wait()

    iota = lax.iota(jnp.int32, 16)
    zero = iota * 0
    i26 = iota * N_CAT
    i13 = iota * N_DENSE

    def chunk(c, carry):
        acc = wb_v[N_CAT + N_DENSE]
        base26 = c * (16 * N_CAT)
        base13 = c * (16 * N_DENSE)
        for f in range(N_CAT):
            v = plsc.load_gather(vals_v, [i26 + (base26 + f), zero])
            acc = acc + v * wb_v[f]
        for d in range(N_DENSE):
            dv = plsc.load_gather(dense_v, [i13 + (base13 + d)])
            acc = acc + dv * wb_v[N_CAT + d]
        c16 = pl.multiple_of(c * 16, 16)
        out_v[pl.ds(c16, 16)] = 1.0 / (1.0 + jnp.exp(-acc))
        return carry

    lax.fori_loop(0, CH, chunk, 0)

    pltpu.sync_copy(out_v, out_hbm.at[pl.ds(wid * ROWS_W, ROWS_W)])


def kernel(cat_indices, dense_features, emb_table, W, b):
    idx_flat = cat_indices.reshape(-1)
    dense_flat = dense_features.reshape(-1)
    wb = jnp.concatenate([W.reshape(-1), b])
    wb_b = jnp.broadcast_to(wb[:, None], (N_CAT + N_DENSE + 1, 16))
    out = _criteo_sc(idx_flat, dense_flat, emb_table, wb_b)
    return out.reshape(BATCH, 1, 1)


# speedup vs baseline: 8.4104x; 8.4104x over previous
import functools

import jax
import jax.numpy as jnp
from jax import lax
from jax.experimental import pallas as pl
from jax.experimental.pallas import tpu as pltpu, tpu_sc as plsc

BATCH = 16384
N_CAT = 26
N_DENSE = 13
NW = 32
ROWS_W = BATCH // NW
IDX_W = ROWS_W * N_CAT
CH = ROWS_W // 16
DEN_W = ROWS_W * N_DENSE

_mesh = plsc.VectorSubcoreMesh(core_axis_name="c", subcore_axis_name="s")


@functools.partial(
    pl.kernel,
    out_type=jax.ShapeDtypeStruct((BATCH,), jnp.float32),
    mesh=_mesh,
    compiler_params=pltpu.CompilerParams(
        use_tc_tiling_on_sc=False, needs_layout_passes=False
    ),
    scratch_types=[
        pltpu.VMEM((IDX_W,), jnp.int32),
        pltpu.VMEM((IDX_W,), jnp.float32),
        pltpu.VMEM((DEN_W,), jnp.float32),
        pltpu.VMEM((N_CAT + N_DENSE + 1, 16), jnp.float32),
        pltpu.VMEM((ROWS_W,), jnp.float32),
        pltpu.SemaphoreType.DMA,
    ],
)
def _criteo_sc(idx_hbm, dense_hbm, table_hbm, wb_hbm, out_hbm,
               idx_v, vals_v, dense_v, wb_v, out_v, sem):
    wid = lax.axis_index("s") * 2 + lax.axis_index("c")

    pltpu.sync_copy(wb_hbm, wb_v)
    pltpu.sync_copy(idx_hbm.at[pl.ds(wid * IDX_W, IDX_W)], idx_v)
    pltpu.sync_copy(dense_hbm.at[pl.ds(wid * DEN_W, DEN_W)], dense_v)

    pltpu.async_copy(table_hbm.at[idx_v], vals_v, sem).wait()

    iota = lax.iota(jnp.int32, 16)
    i26 = iota * N_CAT
    i13 = iota * N_DENSE

    def chunk(c, carry):
        acc = wb_v[N_CAT + N_DENSE]
        base26 = c * (16 * N_CAT)
        base13 = c * (16 * N_DENSE)
        for f in range(N_CAT):
            v = plsc.load_gather(vals_v, [i26 + (base26 + f)])
            acc = acc + v * wb_v[f]
        for d in range(N_DENSE):
            dv = plsc.load_gather(dense_v, [i13 + (base13 + d)])
            acc = acc + dv * wb_v[N_CAT + d]
        c16 = pl.multiple_of(c * 16, 16)
        out_v[pl.ds(c16, 16)] = 1.0 / (1.0 + jnp.exp(-acc))
        return carry

    lax.fori_loop(0, CH, chunk, 0)

    pltpu.sync_copy(out_v, out_hbm.at[pl.ds(wid * ROWS_W, ROWS_W)])


def kernel(cat_indices, dense_features, emb_table, W, b):
    idx_flat = cat_indices.reshape(-1)
    dense_flat = dense_features.reshape(-1)
    wb = jnp.concatenate([W.reshape(-1), b])
    wb_b = jnp.broadcast_to(wb[:, None], (N_CAT + N_DENSE + 1, 16))
    out = _criteo_sc(idx_flat, dense_flat, emb_table[:, 0], wb_b)
    return out.reshape(BATCH, 1, 1)


# speedup vs baseline: 10.8563x; 1.2908x over previous
import functools

import jax
import jax.numpy as jnp
from jax import lax
from jax.experimental import pallas as pl
from jax.experimental.pallas import tpu as pltpu, tpu_sc as plsc

BATCH = 16384
N_CAT = 26
N_DENSE = 13
NW = 32
ROWS_W = BATCH // NW
IDX_W = ROWS_W * N_CAT
CH = ROWS_W // 16
DEN_W = ROWS_W * N_DENSE
VOCAB = 1000000
SPLIT = 2
SHARD = VOCAB // SPLIT

_mesh = plsc.VectorSubcoreMesh(core_axis_name="c", subcore_axis_name="s")


@functools.partial(
    pl.kernel,
    out_type=jax.ShapeDtypeStruct((BATCH,), jnp.float32),
    mesh=_mesh,
    compiler_params=pltpu.CompilerParams(
        use_tc_tiling_on_sc=False, needs_layout_passes=False
    ),
    scratch_types=[
        [pltpu.VMEM((IDX_W,), jnp.int32) for _ in range(SPLIT)],
        pltpu.VMEM((IDX_W,), jnp.float32),
        pltpu.VMEM((DEN_W,), jnp.float32),
        pltpu.VMEM((N_CAT + N_DENSE + 1, 16), jnp.float32),
        pltpu.VMEM((ROWS_W,), jnp.float32),
        pltpu.SemaphoreType.DMA,
        pltpu.SemaphoreType.DMA,
        pltpu.SemaphoreType.DMA,
    ],
)
def _criteo_sc(idx0_hbm, idx1_hbm, dense_hbm,
               t0_hbm, t1_hbm, wb_hbm, out_hbm,
               idx_vs, vals_v, dense_v, wb_v, out_v, sem_i, sem_s, sem_g):
    wid = lax.axis_index("s") * 2 + lax.axis_index("c")

    idx_hbms = (idx0_hbm, idx1_hbm)
    t_hbms = (t0_hbm, t1_hbm)
    cp_is = [
        pltpu.async_copy(h.at[pl.ds(wid * IDX_W, IDX_W)], v, sem_i)
        for h, v in zip(idx_hbms, idx_vs)
    ]
    cp_w = pltpu.async_copy(wb_hbm, wb_v, sem_s)
    cp_d = pltpu.async_copy(dense_hbm.at[pl.ds(wid * DEN_W, DEN_W)], dense_v, sem_s)

    cp_gs = []
    for k in range(SPLIT):
        cp_is[k].wait()
        cp_gs.append(
            pltpu.async_copy(
                t_hbms[k].at[plsc.Indices(idx_vs[k], ignored_value=-1)],
                vals_v,
                sem_g,
            )
        )
    cp_w.wait()
    cp_d.wait()
    for cp in cp_gs:
        cp.wait()

    def chunk(c, carry):
        c16 = pl.multiple_of(c * 16, 16)
        acc = wb_v[N_CAT + N_DENSE]
        for f in range(N_CAT):
            acc = acc + vals_v[pl.ds(f * ROWS_W + c16, 16)] * wb_v[f]
        for d in range(N_DENSE):
            acc = acc + dense_v[pl.ds(d * ROWS_W + c16, 16)] * wb_v[N_CAT + d]
        out_v[pl.ds(c16, 16)] = 1.0 / (1.0 + jnp.exp(-acc))
        return carry

    lax.fori_loop(0, CH, chunk, 0)

    pltpu.sync_copy(out_v, out_hbm.at[pl.ds(wid * ROWS_W, ROWS_W)])


def kernel(cat_indices, dense_features, emb_table, W, b):
    idx_t = cat_indices.reshape(NW, ROWS_W, N_CAT).transpose(0, 2, 1).reshape(-1)
    dense_flat = dense_features.reshape(NW, ROWS_W, N_DENSE).transpose(0, 2, 1).reshape(-1)
    idx_parts = [
        jnp.where(
            (idx_t >= k * SHARD) & (idx_t < (k + 1) * SHARD), idx_t - k * SHARD, -1
        )
        for k in range(SPLIT)
    ]
    tables = [emb_table[k * SHARD:(k + 1) * SHARD, 0] for k in range(SPLIT)]
    wb = jnp.concatenate([W.reshape(-1), b])
    wb_b = jnp.broadcast_to(wb[:, None], (N_CAT + N_DENSE + 1, 16))
    out = _criteo_sc(*idx_parts, dense_flat, *tables, wb_b)
    return out.reshape(BATCH, 1, 1)
